# baseline (speedup 1.0000x reference)
import dataclasses
import functools

import jax
import jax.numpy as jnp
from jax import lax
from jax.experimental import pallas as pl
from jax.experimental.pallas import tpu as pltpu
from jax.experimental.pallas import tpu_sc as plsc

N_NODES = 10000
N_EDGES = 320000
D_FEAT = 128
D_OUT = 64
BATCH = 1024

NC = 2
NS = 16
NW = NC * NS
CH = 128
N_PAD = 10240
ROWS_PER_TILE = N_PAD // NS
E_PAD = 327680
CHUNKS_PER_TILE = E_PAD // (NW * CH)
HROWS_PER_TILE = E_PAD // (NW * 16)

_mesh = plsc.VectorSubcoreMesh(core_axis_name="c", subcore_axis_name="s")


def _sc_params():
    cp = pltpu.CompilerParams()
    if "needs_layout_passes" in pltpu.CompilerParams.__dataclass_fields__:
        cp = dataclasses.replace(cp, needs_layout_passes=False)
    return cp


@functools.partial(
    pl.kernel,
    out_type=jax.ShapeDtypeStruct((NC, N_PAD), jnp.float32),
    mesh=_mesh,
    compiler_params=_sc_params(),
    scratch_types=[
        pltpu.VMEM((HROWS_PER_TILE, 16), jnp.int32),
        pltpu.VMEM((N_PAD,), jnp.float32),
        pltpu.VMEM((NS, ROWS_PER_TILE), jnp.float32),
        pltpu.VMEM((ROWS_PER_TILE,), jnp.float32),
        pltpu.VMEM_SHARED((NS, N_PAD), jnp.float32),
    ],
)
def _sc_degree(dst_hbm, zeros_hbm, out_hbm, dst_v, hist_v, colsum_v, deg_v,
               hist_sh):
    cid = lax.axis_index("c")
    sid = lax.axis_index("s")
    w2 = cid * NS + sid
    pltpu.sync_copy(zeros_hbm, hist_v)
    pltpu.sync_copy(dst_hbm.at[pl.ds(w2 * HROWS_PER_TILE, HROWS_PER_TILE)],
                    dst_v)

    @pl.loop(0, HROWS_PER_TILE)
    def _(j):
        d16 = dst_v[j]
        cnt, last = plsc.scan_count(d16)
        plsc.addupdate_scatter(hist_v, [d16], cnt.astype(jnp.float32),
                               mask=last)

    pltpu.sync_copy(hist_v, hist_sh.at[sid])
    plsc.subcore_barrier()
    pltpu.sync_copy(hist_sh.at[:, pl.ds(sid * ROWS_PER_TILE, ROWS_PER_TILE)],
                    colsum_v)

    @pl.loop(0, ROWS_PER_TILE // 16)
    def _(cc):
        acc = colsum_v[0, pl.ds(cc * 16, 16)]
        for t in range(1, NS):
            acc = acc + colsum_v[t, pl.ds(cc * 16, 16)]
        deg_v[pl.ds(cc * 16, 16)] = acc

    pltpu.sync_copy(deg_v,
                    out_hbm.at[cid, pl.ds(sid * ROWS_PER_TILE, ROWS_PER_TILE)])


PH = 32
C0_CHUNKS = 512
C1_CHUNKS = 2048
TILE_C0 = C0_CHUNKS // NS
TILE_C1 = C1_CHUNKS // NS
MAX_PHASES = max(TILE_C0, TILE_C1) // PH


@functools.partial(
    pl.kernel,
    out_type=jax.ShapeDtypeStruct((NC, N_PAD, D_FEAT), jnp.float32),
    mesh=_mesh,
    scratch_types=[
        pltpu.VMEM((PH, CH), jnp.int32),
        pltpu.VMEM((PH, CH), jnp.int32),
        pltpu.VMEM((CH, D_FEAT), jnp.float32),
        pltpu.VMEM((CH, D_FEAT), jnp.float32),
        pltpu.VMEM_SHARED((N_PAD, D_FEAT), jnp.float32),
        pltpu.SemaphoreType.DMA,
        pltpu.SemaphoreType.DMA,
        pltpu.SemaphoreType.DMA,
        pltpu.SemaphoreType.DMA,
    ],
)
def _sc_propagate(h_hbm, src_hbm, dst_hbm, zeros_hbm, out_hbm, src_v, dst_v,
                  rows_a, rows_b, acc_sh, sem_ga, sem_gb, sem_sa, sem_sb):
    cid = lax.axis_index("c")
    sid = lax.axis_index("s")
    pltpu.sync_copy(zeros_hbm, rows_a)

    @pl.loop(0, ROWS_PER_TILE // CH)
    def _(k):
        pltpu.sync_copy(rows_a,
                        acc_sh.at[pl.ds(sid * ROWS_PER_TILE + k * CH, CH)])

    plsc.subcore_barrier()

    my_n = jnp.where(cid == 0, TILE_C0, TILE_C1)
    my_base = jnp.where(cid == 0, sid * TILE_C0, C0_CHUNKS + sid * TILE_C1)

    @pl.loop(0, MAX_PHASES)
    def _(ph):
        @pl.when(ph * PH < my_n)
        def _():
            base = my_base + ph * PH
            pltpu.sync_copy(src_hbm.at[pl.ds(base, PH)], src_v)
            pltpu.sync_copy(dst_hbm.at[pl.ds(base, PH)], dst_v)
            pltpu.async_copy(h_hbm.at[src_v.at[0]], rows_a, sem_ga)
            pltpu.async_copy(h_hbm.at[src_v.at[1]], rows_b, sem_gb)

            @pl.loop(0, PH, step=2)
            def _(g):
                pltpu.make_async_copy(h_hbm.at[src_v.at[g]], rows_a,
                                      sem_ga).wait()
                sca = pltpu.async_copy(rows_a, acc_sh.at[dst_v.at[g]], sem_sa,
                                       add=True)
                pltpu.make_async_copy(h_hbm.at[src_v.at[g + 1]], rows_b,
                                      sem_gb).wait()
                scb = pltpu.async_copy(rows_b, acc_sh.at[dst_v.at[g + 1]],
                                       sem_sb, add=True)
                sca.wait()

                @pl.when(g + 2 < PH)
                def _():
                    pltpu.async_copy(h_hbm.at[src_v.at[g + 2]], rows_a, sem_ga)

                scb.wait()

                @pl.when(g + 3 < PH)
                def _():
                    pltpu.async_copy(h_hbm.at[src_v.at[g + 3]], rows_b, sem_gb)

    plsc.subcore_barrier()
    pltpu.sync_copy(acc_sh.at[pl.ds(sid * ROWS_PER_TILE, ROWS_PER_TILE)],
                    out_hbm.at[cid, pl.ds(sid * ROWS_PER_TILE, ROWS_PER_TILE)])


@functools.partial(
    pl.kernel,
    out_type=jax.ShapeDtypeStruct((BATCH, D_FEAT), jnp.float32),
    mesh=_mesh,
    scratch_types=[
        pltpu.VMEM((BATCH // NW,), jnp.int32),
        pltpu.VMEM((BATCH // NW, D_FEAT), jnp.float32),
        pltpu.SemaphoreType.DMA,
    ],
)
def _sc_gather_rows(table_hbm, idx_hbm, out_hbm, idx_v, rows_v, sem):
    b_per_w = BATCH // NW
    base = (lax.axis_index("s") * NC + lax.axis_index("c")) * b_per_w
    pltpu.sync_copy(idx_hbm.at[pl.ds(base, b_per_w)], idx_v)
    pltpu.async_copy(table_hbm.at[idx_v], rows_v, sem).wait()
    pltpu.sync_copy(rows_v, out_hbm.at[pl.ds(base, b_per_w)])


def _tc_scale_body(x_ref, deg_ref, y_ref):
    d = deg_ref[0, :N_NODES, :] + deg_ref[1, :N_NODES, :]
    dis = lax.rsqrt(jnp.maximum(d, 1.0))
    y_ref[...] = x_ref[...] * dis


def _tc_mid_body(p_ref, deg_ref, w_ref, z_ref):
    s = p_ref[0, :N_NODES, :] + p_ref[1, :N_NODES, :]
    d = deg_ref[0, :N_NODES, :] + deg_ref[1, :N_NODES, :]
    dis = lax.rsqrt(jnp.maximum(d, 1.0))
    h = lax.dot_general(s * dis, w_ref[...], (((1,), (0,)), ((), ())),
                        precision=lax.Precision.HIGHEST)
    z_ref[...] = jnp.maximum(h, 0.0) * dis


def _tc_out_body(q_ref, deg_ref, w_ref, o_ref):
    s = q_ref[0, :N_NODES, :] + q_ref[1, :N_NODES, :]
    d = deg_ref[0, :N_NODES, :] + deg_ref[1, :N_NODES, :]
    dis = lax.rsqrt(jnp.maximum(d, 1.0))
    o_ref[...] = lax.dot_general(s * dis, w_ref[...], (((1,), (0,)), ((), ())),
                                 precision=lax.Precision.HIGHEST)


def kernel(x, W1, W2, edge_index, idx):
    src = edge_index[0]
    dst = edge_index[1]
    pad = E_PAD - N_EDGES
    src_p = jnp.concatenate([src, jnp.zeros((pad,), jnp.int32)])
    dst_p = jnp.concatenate([dst, jnp.full((pad,), N_NODES, jnp.int32)])
    src2d = src_p.reshape(E_PAD // CH, CH)
    dst2d = dst_p.reshape(E_PAD // CH, CH)
    dst16 = dst_p.reshape(E_PAD // 16, 16)

    zeros_flat = jnp.zeros((N_PAD,), jnp.float32)
    zeros_blk = jnp.zeros((CH, D_FEAT), jnp.float32)
    W2p = jnp.concatenate(
        [W2, jnp.zeros((D_FEAT, D_FEAT - D_OUT), jnp.float32)], axis=1)

    deg = _sc_degree(dst16, zeros_flat)
    deg_col = deg.reshape(NC, N_PAD, 1)

    y = pl.pallas_call(
        _tc_scale_body,
        out_shape=jax.ShapeDtypeStruct((N_NODES, D_FEAT), jnp.float32),
    )(x, deg_col)

    p = _sc_propagate(y, src2d, dst2d, zeros_blk)

    z = pl.pallas_call(
        _tc_mid_body,
        out_shape=jax.ShapeDtypeStruct((N_NODES, D_FEAT), jnp.float32),
    )(p, deg_col, W1)

    q = _sc_propagate(z, src2d, dst2d, zeros_blk)

    out_full = pl.pallas_call(
        _tc_out_body,
        out_shape=jax.ShapeDtypeStruct((N_NODES, D_FEAT), jnp.float32),
    )(q, deg_col, W2p)

    return _sc_gather_rows(out_full, idx)[:, :D_OUT]


# speedup vs baseline: 1.7795x; 1.7795x over previous
import dataclasses
import functools

import jax
import jax.numpy as jnp
from jax import lax
from jax.experimental import pallas as pl
from jax.experimental.pallas import tpu as pltpu
from jax.experimental.pallas import tpu_sc as plsc

N_NODES = 10000
N_EDGES = 320000
D_FEAT = 128
D_OUT = 64
BATCH = 1024

NC = 2
NS = 16
NW = NC * NS
CH = 128
N_PAD = 10240
ROWS_PER_TILE = N_PAD // NS
E_PAD = 327680
CHUNKS_PER_TILE = E_PAD // (NW * CH)
HROWS_PER_TILE = E_PAD // (NW * 16)

_mesh = plsc.VectorSubcoreMesh(core_axis_name="c", subcore_axis_name="s")


def _sc_params():
    cp = pltpu.CompilerParams()
    if "needs_layout_passes" in pltpu.CompilerParams.__dataclass_fields__:
        cp = dataclasses.replace(cp, needs_layout_passes=False)
    return cp


@functools.partial(
    pl.kernel,
    out_type=(
        jax.ShapeDtypeStruct((NC, N_PAD), jnp.float32),
        jax.ShapeDtypeStruct((NC * BATCH,), jnp.float32),
    ),
    mesh=_mesh,
    compiler_params=_sc_params(),
    scratch_types=[
        pltpu.VMEM((E_PAD // NW,), jnp.int32),
        pltpu.VMEM((N_PAD,), jnp.float32),
        pltpu.VMEM((NS, ROWS_PER_TILE), jnp.float32),
        pltpu.VMEM((ROWS_PER_TILE,), jnp.float32),
        pltpu.VMEM((BATCH // NS,), jnp.int32),
        pltpu.VMEM((BATCH // NS,), jnp.float32),
        pltpu.VMEM_SHARED((NS, N_PAD), jnp.float32),
    ],
)
def _sc_degree(dst_hbm, idx_hbm, zeros_hbm, out_hbm, degb_hbm, dst_v, hist_v,
               colsum_v, deg_v, idxb_v, degb_v, hist_sh):
    cid = lax.axis_index("c")
    sid = lax.axis_index("s")
    w2 = cid * NS + sid
    b_per_t = BATCH // NS
    ET = E_PAD // NW
    pltpu.sync_copy(zeros_hbm, hist_v)
    pltpu.sync_copy(dst_hbm.at[pl.ds(w2 * ET, ET)], dst_v)
    pltpu.sync_copy(idx_hbm.at[pl.ds(sid * b_per_t, b_per_t)], idxb_v)

    @pl.loop(0, ET // 16)
    def _(j):
        d16 = dst_v[pl.ds(j * 16, 16)]
        cnt, last = plsc.scan_count(d16)
        plsc.addupdate_scatter(hist_v, [d16], cnt.astype(jnp.float32),
                               mask=last)

    pltpu.sync_copy(hist_v, hist_sh.at[sid])
    plsc.subcore_barrier()
    pltpu.sync_copy(hist_sh.at[:, pl.ds(sid * ROWS_PER_TILE, ROWS_PER_TILE)],
                    colsum_v)

    @pl.loop(0, ROWS_PER_TILE // 16)
    def _(cc):
        acc = colsum_v[0, pl.ds(cc * 16, 16)]
        for t in range(1, NS):
            acc = acc + colsum_v[t, pl.ds(cc * 16, 16)]
        deg_v[pl.ds(cc * 16, 16)] = acc

    pltpu.sync_copy(deg_v,
                    out_hbm.at[cid, pl.ds(sid * ROWS_PER_TILE, ROWS_PER_TILE)])

    plsc.subcore_barrier()
    pltpu.sync_copy(deg_v, hist_sh.at[0, pl.ds(sid * ROWS_PER_TILE,
                                               ROWS_PER_TILE)])
    plsc.subcore_barrier()
    pltpu.sync_copy(hist_sh.at[0], hist_v)

    @pl.loop(0, b_per_t // 16)
    def _(r2):
        iv = idxb_v[pl.ds(r2 * 16, 16)]
        degb_v[pl.ds(r2 * 16, 16)] = plsc.load_gather(hist_v, [iv])

    pltpu.sync_copy(degb_v,
                    degb_hbm.at[pl.ds(cid * BATCH + sid * b_per_t, b_per_t)])


PH_CHUNKS = CHUNKS_PER_TILE // 2


@functools.partial(
    pl.kernel,
    out_type=jax.ShapeDtypeStruct((NC, N_PAD, D_FEAT), jnp.float32),
    mesh=_mesh,
    scratch_types=[
        pltpu.VMEM((PH_CHUNKS, CH), jnp.int32),
        pltpu.VMEM((PH_CHUNKS, CH), jnp.int32),
        pltpu.VMEM((CH, D_FEAT), jnp.float32),
        pltpu.VMEM((CH, D_FEAT), jnp.float32),
        pltpu.VMEM_SHARED((N_PAD, D_FEAT), jnp.float32),
        pltpu.SemaphoreType.DMA,
        pltpu.SemaphoreType.DMA,
        pltpu.SemaphoreType.DMA,
        pltpu.SemaphoreType.DMA,
    ],
)
def _sc_propagate(h_hbm, src_hbm, dst_hbm, zeros_hbm, out_hbm, src_v, dst_v,
                  rows_a, rows_b, acc_sh, sem_ga, sem_gb, sem_sa, sem_sb):
    cid = lax.axis_index("c")
    sid = lax.axis_index("s")
    wid = sid * NC + cid
    pltpu.sync_copy(zeros_hbm, rows_a)

    @pl.loop(0, ROWS_PER_TILE // CH)
    def _(k):
        pltpu.sync_copy(rows_a,
                        acc_sh.at[pl.ds(sid * ROWS_PER_TILE + k * CH, CH)])

    plsc.subcore_barrier()

    for ph in range(2):
        base = wid * CHUNKS_PER_TILE + ph * PH_CHUNKS
        pltpu.sync_copy(src_hbm.at[pl.ds(base, PH_CHUNKS)], src_v)
        pltpu.sync_copy(dst_hbm.at[pl.ds(base, PH_CHUNKS)], dst_v)
        pltpu.async_copy(h_hbm.at[src_v.at[0]], rows_a, sem_ga)
        pltpu.async_copy(h_hbm.at[src_v.at[1]], rows_b, sem_gb)

        @pl.loop(0, PH_CHUNKS, step=2)
        def _(g):
            pltpu.make_async_copy(h_hbm.at[src_v.at[g]], rows_a, sem_ga).wait()
            sca = pltpu.async_copy(rows_a, acc_sh.at[dst_v.at[g]], sem_sa,
                                   add=True)
            pltpu.make_async_copy(h_hbm.at[src_v.at[g + 1]], rows_b,
                                  sem_gb).wait()
            scb = pltpu.async_copy(rows_b, acc_sh.at[dst_v.at[g + 1]], sem_sb,
                                   add=True)
            sca.wait()

            @pl.when(g + 2 < PH_CHUNKS)
            def _():
                pltpu.async_copy(h_hbm.at[src_v.at[g + 2]], rows_a, sem_ga)

            scb.wait()

            @pl.when(g + 3 < PH_CHUNKS)
            def _():
                pltpu.async_copy(h_hbm.at[src_v.at[g + 3]], rows_b, sem_gb)

    plsc.subcore_barrier()
    pltpu.sync_copy(acc_sh.at[pl.ds(sid * ROWS_PER_TILE, ROWS_PER_TILE)],
                    out_hbm.at[cid, pl.ds(sid * ROWS_PER_TILE, ROWS_PER_TILE)])


KCAP_CHUNKS = 88
KCAP = KCAP_CHUNKS * CH
C_PAD = 1152
CROWS_PER_TILE = C_PAD // NS


@functools.partial(
    pl.kernel,
    out_type=(
        jax.ShapeDtypeStruct((NW, KCAP_CHUNKS, CH), jnp.int32),
        jax.ShapeDtypeStruct((NW, KCAP_CHUNKS, CH), jnp.int32),
        jax.ShapeDtypeStruct((NW * 16,), jnp.int32),
        jax.ShapeDtypeStruct((NW * 32,), jnp.int32),
    ),
    mesh=_mesh,
    compiler_params=_sc_params(),
    scratch_types=[
        pltpu.VMEM((2 * E_PAD // NW,), jnp.int32),
        pltpu.VMEM((N_PAD,), jnp.int32),
        pltpu.VMEM((2 * KCAP_CHUNKS, CH), jnp.int32),
        pltpu.VMEM((BATCH + 64,), jnp.int32),
    ],
)
def _sc_filter(dst_hbm, src_hbm, idx_hbm, sentinv_hbm, sentk_hbm,
               srcK_hbm, posK_hbm, cnts_hbm, posB_hbm,
               sd_v, inv_v, spk_v, misc_v):
    cid = lax.axis_index("c")
    sid = lax.axis_index("s")
    w2 = cid * NS + sid
    ET = E_PAD // NW
    pltpu.sync_copy(dst_hbm.at[pl.ds(w2 * ET, ET)], sd_v.at[pl.ds(0, ET)])
    pltpu.sync_copy(src_hbm.at[pl.ds(w2 * ET, ET)], sd_v.at[pl.ds(ET, ET)])
    pltpu.sync_copy(idx_hbm, misc_v.at[pl.ds(0, BATCH)])
    pltpu.sync_copy(sentinv_hbm, inv_v)
    pltpu.sync_copy(sentk_hbm, spk_v)

    @pl.loop(0, BATCH // 16)
    def _(r):
        v16 = misc_v[pl.ds(r * 16, 16)]
        _, last = plsc.scan_count(v16)
        b16 = r * 16 + lax.iota(jnp.int32, 16)
        plsc.store_scatter(inv_v, [v16], b16, mask=last)

    def body(j, cnt16):
        d16 = sd_v[pl.ds(j * 16, 16)]
        s16 = sd_v[pl.ds(ET + j * 16, 16)]
        p16 = plsc.load_gather(inv_v, [d16])
        m = p16 < BATCH
        offs = plsc.cumsum(m.astype(jnp.int32))
        tgt = cnt16 + offs - 1
        tr = tgt // CH
        tc = tgt - tr * CH
        plsc.store_scatter(spk_v, [tr, tc], s16, mask=m)
        plsc.store_scatter(spk_v, [tr + KCAP_CHUNKS, tc], p16, mask=m)
        return cnt16 + plsc.all_reduce_population_count(m)

    cnt16 = lax.fori_loop(0, ET // 16, body, jnp.zeros((16,), jnp.int32))

    @pl.loop(0, 2)
    def _(r2):
        iv = misc_v[pl.ds(w2 * 32 + r2 * 16, 16)]
        misc_v[pl.ds(BATCH + r2 * 16, 16)] = plsc.load_gather(inv_v, [iv])

    misc_v[pl.ds(BATCH + 32, 16)] = cnt16
    pltpu.sync_copy(spk_v.at[pl.ds(0, KCAP_CHUNKS)], srcK_hbm.at[w2])
    pltpu.sync_copy(spk_v.at[pl.ds(KCAP_CHUNKS, KCAP_CHUNKS)],
                    posK_hbm.at[w2])
    pltpu.sync_copy(misc_v.at[pl.ds(BATCH + 32, 16)],
                    cnts_hbm.at[pl.ds(w2 * 16, 16)])
    pltpu.sync_copy(misc_v.at[pl.ds(BATCH, 32)],
                    posB_hbm.at[pl.ds(w2 * 32, 32)])


@functools.partial(
    pl.kernel,
    out_type=jax.ShapeDtypeStruct((NC, C_PAD, D_FEAT), jnp.float32),
    mesh=_mesh,
    compiler_params=_sc_params(),
    scratch_types=[
        pltpu.VMEM((KCAP_CHUNKS, CH), jnp.int32),
        pltpu.VMEM((KCAP_CHUNKS, CH), jnp.int32),
        pltpu.VMEM((16,), jnp.int32),
        pltpu.VMEM((CH, D_FEAT), jnp.float32),
        pltpu.VMEM((CH, D_FEAT), jnp.float32),
        pltpu.VMEM_SHARED((C_PAD, D_FEAT), jnp.float32),
        pltpu.SemaphoreType.DMA,
        pltpu.SemaphoreType.DMA,
        pltpu.SemaphoreType.DMA,
        pltpu.SemaphoreType.DMA,
    ],
)
def _sc_prop_compact(h_hbm, srcK_hbm, posK_hbm, cnts_hbm, zeros_hbm, out_hbm,
                     srcK_v, posK_v, cnt_v, rows_a, rows_b, acc_sh, sem_ga,
                     sem_gb, sem_sa, sem_sb):
    cid = lax.axis_index("c")
    sid = lax.axis_index("s")
    w2 = cid * NS + sid
    pltpu.sync_copy(zeros_hbm.at[pl.ds(0, CROWS_PER_TILE)],
                    acc_sh.at[pl.ds(sid * CROWS_PER_TILE, CROWS_PER_TILE)])
    pltpu.sync_copy(srcK_hbm.at[w2], srcK_v)
    pltpu.sync_copy(posK_hbm.at[w2], posK_v)
    pltpu.sync_copy(cnts_hbm.at[pl.ds(w2 * 16, 16)], cnt_v)
    plsc.subcore_barrier()

    cnt = jnp.max(cnt_v[...])

    def active(g):
        return g * CH < cnt

    @pl.when(active(0))
    def _():
        pltpu.async_copy(h_hbm.at[srcK_v.at[0]], rows_a, sem_ga)

    @pl.when(active(1))
    def _():
        pltpu.async_copy(h_hbm.at[srcK_v.at[1]], rows_b, sem_gb)

    @pl.loop(0, KCAP_CHUNKS, step=2)
    def _(g):
        @pl.when(active(g))
        def _():
            pltpu.make_async_copy(h_hbm.at[srcK_v.at[g]], rows_a,
                                  sem_ga).wait()
            sca = pltpu.async_copy(rows_a, acc_sh.at[posK_v.at[g]], sem_sa,
                                   add=True)
            sca.wait()

            @pl.when(active(g + 2))
            def _():
                pltpu.async_copy(h_hbm.at[srcK_v.at[g + 2]], rows_a, sem_ga)

        @pl.when(active(g + 1))
        def _():
            pltpu.make_async_copy(h_hbm.at[srcK_v.at[g + 1]], rows_b,
                                  sem_gb).wait()
            scb = pltpu.async_copy(rows_b, acc_sh.at[posK_v.at[g + 1]], sem_sb,
                                   add=True)
            scb.wait()

            @pl.when(active(g + 3))
            def _():
                pltpu.async_copy(h_hbm.at[srcK_v.at[g + 3]], rows_b, sem_gb)

    plsc.subcore_barrier()
    pltpu.sync_copy(acc_sh.at[pl.ds(sid * CROWS_PER_TILE, CROWS_PER_TILE)],
                    out_hbm.at[cid, pl.ds(sid * CROWS_PER_TILE,
                                          CROWS_PER_TILE)])


@functools.partial(
    pl.kernel,
    out_type=jax.ShapeDtypeStruct((BATCH, D_FEAT), jnp.float32),
    mesh=_mesh,
    scratch_types=[
        pltpu.VMEM((BATCH // NW,), jnp.int32),
        pltpu.VMEM((BATCH // NW, D_FEAT), jnp.float32),
        pltpu.SemaphoreType.DMA,
    ],
)
def _sc_gather_rows(table_hbm, idx_hbm, out_hbm, idx_v, rows_v, sem):
    b_per_w = BATCH // NW
    base = (lax.axis_index("s") * NC + lax.axis_index("c")) * b_per_w
    pltpu.sync_copy(idx_hbm.at[pl.ds(base, b_per_w)], idx_v)
    pltpu.async_copy(table_hbm.at[idx_v], rows_v, sem).wait()
    pltpu.sync_copy(rows_v, out_hbm.at[pl.ds(base, b_per_w)])


def _tc_scale_body(x_ref, deg_ref, y_ref):
    d = deg_ref[0, :N_NODES, :] + deg_ref[1, :N_NODES, :]
    dis = lax.rsqrt(jnp.maximum(d, 1.0))
    y_ref[...] = x_ref[...] * dis


def _tc_mid_body(p_ref, deg_ref, w_ref, z_ref):
    s = p_ref[0, :N_NODES, :] + p_ref[1, :N_NODES, :]
    d = deg_ref[0, :N_NODES, :] + deg_ref[1, :N_NODES, :]
    dis = lax.rsqrt(jnp.maximum(d, 1.0))
    h = lax.dot_general(s * dis, w_ref[...], (((1,), (0,)), ((), ())),
                        precision=lax.Precision.HIGHEST)
    z_ref[...] = jnp.maximum(h, 0.0) * dis


def _tc_out_body(c_ref, degb_ref, w_ref, o_ref):
    s = c_ref[0, :BATCH, :] + c_ref[1, :BATCH, :]
    d = degb_ref[0] + degb_ref[1]
    dis = lax.rsqrt(jnp.maximum(d, 1.0))
    o_ref[...] = lax.dot_general(s * dis, w_ref[...], (((1,), (0,)), ((), ())),
                                 precision=lax.Precision.HIGHEST)


def kernel(x, W1, W2, edge_index, idx):
    src = edge_index[0]
    dst = edge_index[1]
    pad = E_PAD - N_EDGES
    src_p = jnp.concatenate([src, jnp.zeros((pad,), jnp.int32)])
    dst_p = jnp.concatenate([dst, jnp.full((pad,), N_NODES, jnp.int32)])
    src2d = src_p.reshape(E_PAD // CH, CH)
    dst2d = dst_p.reshape(E_PAD // CH, CH)

    zeros_flat = jnp.zeros((N_PAD,), jnp.float32)
    zeros_blk = jnp.zeros((CH, D_FEAT), jnp.float32)
    sentinv = jnp.full((N_PAD,), BATCH, jnp.int32)
    sentk = jnp.concatenate([jnp.zeros((KCAP_CHUNKS, CH), jnp.int32),
                             jnp.full((KCAP_CHUNKS, CH), BATCH, jnp.int32)])
    W2p = jnp.concatenate(
        [W2, jnp.zeros((D_FEAT, D_FEAT - D_OUT), jnp.float32)], axis=1)

    deg, degB = _sc_degree(dst_p, idx, zeros_flat)
    deg_col = deg.reshape(NC, N_PAD, 1)

    srcK, posK, cnts, posB = _sc_filter(dst_p, src_p, idx, sentinv, sentk)
    posB = posB.reshape(BATCH)

    y = pl.pallas_call(
        _tc_scale_body,
        out_shape=jax.ShapeDtypeStruct((N_NODES, D_FEAT), jnp.float32),
    )(x, deg_col)

    p = _sc_propagate(y, src2d, dst2d, zeros_blk)

    z = pl.pallas_call(
        _tc_mid_body,
        out_shape=jax.ShapeDtypeStruct((N_NODES, D_FEAT), jnp.float32),
    )(p, deg_col, W1)

    c = _sc_prop_compact(z, srcK, posK, cnts, zeros_blk)

    out_c = pl.pallas_call(
        _tc_out_body,
        out_shape=jax.ShapeDtypeStruct((BATCH, D_FEAT), jnp.float32),
    )(c, degB.reshape(NC, BATCH, 1), W2p)

    return _sc_gather_rows(out_c, posB)[:, :D_OUT]


# speedup vs baseline: 1.7853x; 1.0033x over previous
import dataclasses
import functools

import jax
import jax.numpy as jnp
from jax import lax
from jax.experimental import pallas as pl
from jax.experimental.pallas import tpu as pltpu
from jax.experimental.pallas import tpu_sc as plsc

N_NODES = 10000
N_EDGES = 320000
D_FEAT = 128
D_OUT = 64
BATCH = 1024

NC = 2
NS = 16
NW = NC * NS
CH = 128
N_PAD = 10240
ROWS_PER_TILE = N_PAD // NS
E_PAD = 327680
CHUNKS_PER_TILE = E_PAD // (NW * CH)
HROWS_PER_TILE = E_PAD // (NW * 16)

_mesh = plsc.VectorSubcoreMesh(core_axis_name="c", subcore_axis_name="s")


def _sc_params():
    cp = pltpu.CompilerParams()
    if "needs_layout_passes" in pltpu.CompilerParams.__dataclass_fields__:
        cp = dataclasses.replace(cp, needs_layout_passes=False)
    return cp


@functools.partial(
    pl.kernel,
    out_type=(
        jax.ShapeDtypeStruct((NC, N_PAD), jnp.float32),
        jax.ShapeDtypeStruct((NC * BATCH,), jnp.float32),
    ),
    mesh=_mesh,
    compiler_params=_sc_params(),
    scratch_types=[
        pltpu.VMEM((E_PAD // NW,), jnp.int32),
        pltpu.VMEM((N_PAD,), jnp.float32),
        pltpu.VMEM((NS, ROWS_PER_TILE), jnp.float32),
        pltpu.VMEM((ROWS_PER_TILE,), jnp.float32),
        pltpu.VMEM((BATCH // NS,), jnp.int32),
        pltpu.VMEM((BATCH // NS,), jnp.float32),
        pltpu.VMEM_SHARED((NS, N_PAD), jnp.float32),
    ],
)
def _sc_degree(dst_hbm, idx_hbm, zeros_hbm, out_hbm, degb_hbm, dst_v, hist_v,
               colsum_v, deg_v, idxb_v, degb_v, hist_sh):
    cid = lax.axis_index("c")
    sid = lax.axis_index("s")
    w2 = cid * NS + sid
    b_per_t = BATCH // NS
    ET = E_PAD // NW
    pltpu.sync_copy(zeros_hbm, hist_v)
    pltpu.sync_copy(dst_hbm.at[pl.ds(w2 * ET, ET)], dst_v)
    pltpu.sync_copy(idx_hbm.at[pl.ds(sid * b_per_t, b_per_t)], idxb_v)

    @pl.loop(0, ET // 16)
    def _(j):
        d16 = dst_v[pl.ds(j * 16, 16)]
        cnt, last = plsc.scan_count(d16)
        plsc.addupdate_scatter(hist_v, [d16], cnt.astype(jnp.float32),
                               mask=last)

    pltpu.sync_copy(hist_v, hist_sh.at[sid])
    plsc.subcore_barrier()
    pltpu.sync_copy(hist_sh.at[:, pl.ds(sid * ROWS_PER_TILE, ROWS_PER_TILE)],
                    colsum_v)

    @pl.loop(0, ROWS_PER_TILE // 16)
    def _(cc):
        acc = colsum_v[0, pl.ds(cc * 16, 16)]
        for t in range(1, NS):
            acc = acc + colsum_v[t, pl.ds(cc * 16, 16)]
        deg_v[pl.ds(cc * 16, 16)] = acc

    pltpu.sync_copy(deg_v,
                    out_hbm.at[cid, pl.ds(sid * ROWS_PER_TILE, ROWS_PER_TILE)])

    plsc.subcore_barrier()
    pltpu.sync_copy(deg_v, hist_sh.at[0, pl.ds(sid * ROWS_PER_TILE,
                                               ROWS_PER_TILE)])
    plsc.subcore_barrier()
    pltpu.sync_copy(hist_sh.at[0], hist_v)

    @pl.loop(0, b_per_t // 16)
    def _(r2):
        iv = idxb_v[pl.ds(r2 * 16, 16)]
        degb_v[pl.ds(r2 * 16, 16)] = plsc.load_gather(hist_v, [iv])

    pltpu.sync_copy(degb_v,
                    degb_hbm.at[pl.ds(cid * BATCH + sid * b_per_t, b_per_t)])


PH_CHUNKS = CHUNKS_PER_TILE // 2


@functools.partial(
    pl.kernel,
    out_type=jax.ShapeDtypeStruct((NC, N_PAD, D_FEAT), jnp.float32),
    mesh=_mesh,
    scratch_types=[
        pltpu.VMEM((PH_CHUNKS, CH), jnp.int32),
        pltpu.VMEM((PH_CHUNKS, CH), jnp.int32),
        pltpu.VMEM((2 * CH, D_FEAT), jnp.float32),
        pltpu.VMEM_SHARED((N_PAD, D_FEAT), jnp.float32),
        pltpu.SemaphoreType.DMA,
        pltpu.SemaphoreType.DMA,
        pltpu.SemaphoreType.DMA,
        pltpu.SemaphoreType.DMA,
        pltpu.SemaphoreType.DMA,
        pltpu.SemaphoreType.DMA,
    ],
)
def _sc_propagate(h_hbm, src_hbm, dst_hbm, zeros_hbm, out_hbm, src_v, dst_v,
                  rows_v, acc_sh, sem_g0, sem_g1, sem_g2, sem_g3, sem_sa,
                  sem_sb):
    cid = lax.axis_index("c")
    sid = lax.axis_index("s")
    wid = sid * NC + cid
    H = CH // 2
    gsem = (sem_g0, sem_g1, sem_g2, sem_g3)
    pltpu.sync_copy(zeros_hbm, rows_v.at[pl.ds(0, CH)])

    @pl.loop(0, ROWS_PER_TILE // CH)
    def _(k):
        pltpu.sync_copy(rows_v.at[pl.ds(0, CH)],
                        acc_sh.at[pl.ds(sid * ROWS_PER_TILE + k * CH, CH)])

    plsc.subcore_barrier()

    def _gather(g, slot):
        for hh in range(2):
            pltpu.async_copy(
                h_hbm.at[src_v.at[g, pl.ds(hh * H, H)]],
                rows_v.at[pl.ds((2 * slot + hh) * H, H)],
                gsem[2 * slot + hh])

    def _gwait(g, slot):
        for hh in range(2):
            pltpu.make_async_copy(
                h_hbm.at[src_v.at[g, pl.ds(hh * H, H)]],
                rows_v.at[pl.ds((2 * slot + hh) * H, H)],
                gsem[2 * slot + hh]).wait()

    for ph in range(2):
        base = wid * CHUNKS_PER_TILE + ph * PH_CHUNKS
        pltpu.sync_copy(src_hbm.at[pl.ds(base, PH_CHUNKS)], src_v)
        pltpu.sync_copy(dst_hbm.at[pl.ds(base, PH_CHUNKS)], dst_v)
        _gather(0, 0)
        _gather(1, 1)

        @pl.loop(0, PH_CHUNKS, step=2)
        def _(g):
            _gwait(g, 0)
            sca = pltpu.async_copy(rows_v.at[pl.ds(0, CH)],
                                   acc_sh.at[dst_v.at[g]], sem_sa, add=True)
            _gwait(g + 1, 1)
            scb = pltpu.async_copy(rows_v.at[pl.ds(CH, CH)],
                                   acc_sh.at[dst_v.at[g + 1]], sem_sb,
                                   add=True)
            sca.wait()

            @pl.when(g + 2 < PH_CHUNKS)
            def _():
                _gather(g + 2, 0)

            scb.wait()

            @pl.when(g + 3 < PH_CHUNKS)
            def _():
                _gather(g + 3, 1)

    plsc.subcore_barrier()
    pltpu.sync_copy(acc_sh.at[pl.ds(sid * ROWS_PER_TILE, ROWS_PER_TILE)],
                    out_hbm.at[cid, pl.ds(sid * ROWS_PER_TILE, ROWS_PER_TILE)])


KCAP_CHUNKS = 88
KCAP = KCAP_CHUNKS * CH
C_PAD = 1152
CROWS_PER_TILE = C_PAD // NS


@functools.partial(
    pl.kernel,
    out_type=(
        jax.ShapeDtypeStruct((NW, KCAP_CHUNKS, CH), jnp.int32),
        jax.ShapeDtypeStruct((NW, KCAP_CHUNKS, CH), jnp.int32),
        jax.ShapeDtypeStruct((NW * 16,), jnp.int32),
        jax.ShapeDtypeStruct((NW * 32,), jnp.int32),
    ),
    mesh=_mesh,
    compiler_params=_sc_params(),
    scratch_types=[
        pltpu.VMEM((2 * E_PAD // NW,), jnp.int32),
        pltpu.VMEM((N_PAD,), jnp.int32),
        pltpu.VMEM((2 * KCAP_CHUNKS, CH), jnp.int32),
        pltpu.VMEM((BATCH + 64,), jnp.int32),
    ],
)
def _sc_filter(dst_hbm, src_hbm, idx_hbm, sentinv_hbm, sentk_hbm,
               srcK_hbm, posK_hbm, cnts_hbm, posB_hbm,
               sd_v, inv_v, spk_v, misc_v):
    cid = lax.axis_index("c")
    sid = lax.axis_index("s")
    w2 = cid * NS + sid
    ET = E_PAD // NW
    pltpu.sync_copy(dst_hbm.at[pl.ds(w2 * ET, ET)], sd_v.at[pl.ds(0, ET)])
    pltpu.sync_copy(src_hbm.at[pl.ds(w2 * ET, ET)], sd_v.at[pl.ds(ET, ET)])
    pltpu.sync_copy(idx_hbm, misc_v.at[pl.ds(0, BATCH)])
    pltpu.sync_copy(sentinv_hbm, inv_v)
    pltpu.sync_copy(sentk_hbm, spk_v)

    @pl.loop(0, BATCH // 16)
    def _(r):
        v16 = misc_v[pl.ds(r * 16, 16)]
        _, last = plsc.scan_count(v16)
        b16 = r * 16 + lax.iota(jnp.int32, 16)
        plsc.store_scatter(inv_v, [v16], b16, mask=last)

    def body(j, cnt16):
        d16 = sd_v[pl.ds(j * 16, 16)]
        s16 = sd_v[pl.ds(ET + j * 16, 16)]
        p16 = plsc.load_gather(inv_v, [d16])
        m = p16 < BATCH
        offs = plsc.cumsum(m.astype(jnp.int32))
        tgt = cnt16 + offs - 1
        tr = tgt // CH
        tc = tgt - tr * CH
        plsc.store_scatter(spk_v, [tr, tc], s16, mask=m)
        plsc.store_scatter(spk_v, [tr + KCAP_CHUNKS, tc], p16, mask=m)
        return cnt16 + plsc.all_reduce_population_count(m)

    cnt16 = lax.fori_loop(0, ET // 16, body, jnp.zeros((16,), jnp.int32))

    @pl.loop(0, 2)
    def _(r2):
        iv = misc_v[pl.ds(w2 * 32 + r2 * 16, 16)]
        misc_v[pl.ds(BATCH + r2 * 16, 16)] = plsc.load_gather(inv_v, [iv])

    misc_v[pl.ds(BATCH + 32, 16)] = cnt16
    pltpu.sync_copy(spk_v.at[pl.ds(0, KCAP_CHUNKS)], srcK_hbm.at[w2])
    pltpu.sync_copy(spk_v.at[pl.ds(KCAP_CHUNKS, KCAP_CHUNKS)],
                    posK_hbm.at[w2])
    pltpu.sync_copy(misc_v.at[pl.ds(BATCH + 32, 16)],
                    cnts_hbm.at[pl.ds(w2 * 16, 16)])
    pltpu.sync_copy(misc_v.at[pl.ds(BATCH, 32)],
                    posB_hbm.at[pl.ds(w2 * 32, 32)])


@functools.partial(
    pl.kernel,
    out_type=jax.ShapeDtypeStruct((NC, C_PAD, D_FEAT), jnp.float32),
    mesh=_mesh,
    compiler_params=_sc_params(),
    scratch_types=[
        pltpu.VMEM((KCAP_CHUNKS, CH), jnp.int32),
        pltpu.VMEM((KCAP_CHUNKS, CH), jnp.int32),
        pltpu.VMEM((16,), jnp.int32),
        pltpu.VMEM((CH, D_FEAT), jnp.float32),
        pltpu.VMEM((CH, D_FEAT), jnp.float32),
        pltpu.VMEM_SHARED((C_PAD, D_FEAT), jnp.float32),
        pltpu.SemaphoreType.DMA,
        pltpu.SemaphoreType.DMA,
        pltpu.SemaphoreType.DMA,
        pltpu.SemaphoreType.DMA,
    ],
)
def _sc_prop_compact(h_hbm, srcK_hbm, posK_hbm, cnts_hbm, zeros_hbm, out_hbm,
                     srcK_v, posK_v, cnt_v, rows_a, rows_b, acc_sh, sem_ga,
                     sem_gb, sem_sa, sem_sb):
    cid = lax.axis_index("c")
    sid = lax.axis_index("s")
    w2 = cid * NS + sid
    pltpu.sync_copy(zeros_hbm.at[pl.ds(0, CROWS_PER_TILE)],
                    acc_sh.at[pl.ds(sid * CROWS_PER_TILE, CROWS_PER_TILE)])
    pltpu.sync_copy(srcK_hbm.at[w2], srcK_v)
    pltpu.sync_copy(posK_hbm.at[w2], posK_v)
    pltpu.sync_copy(cnts_hbm.at[pl.ds(w2 * 16, 16)], cnt_v)
    plsc.subcore_barrier()

    cnt = jnp.max(cnt_v[...])

    def active(g):
        return g * CH < cnt

    @pl.when(active(0))
    def _():
        pltpu.async_copy(h_hbm.at[srcK_v.at[0]], rows_a, sem_ga)

    @pl.when(active(1))
    def _():
        pltpu.async_copy(h_hbm.at[srcK_v.at[1]], rows_b, sem_gb)

    @pl.loop(0, KCAP_CHUNKS, step=2)
    def _(g):
        @pl.when(active(g))
        def _():
            pltpu.make_async_copy(h_hbm.at[srcK_v.at[g]], rows_a,
                                  sem_ga).wait()
            sca = pltpu.async_copy(rows_a, acc_sh.at[posK_v.at[g]], sem_sa,
                                   add=True)
            sca.wait()

            @pl.when(active(g + 2))
            def _():
                pltpu.async_copy(h_hbm.at[srcK_v.at[g + 2]], rows_a, sem_ga)

        @pl.when(active(g + 1))
        def _():
            pltpu.make_async_copy(h_hbm.at[srcK_v.at[g + 1]], rows_b,
                                  sem_gb).wait()
            scb = pltpu.async_copy(rows_b, acc_sh.at[posK_v.at[g + 1]], sem_sb,
                                   add=True)
            scb.wait()

            @pl.when(active(g + 3))
            def _():
                pltpu.async_copy(h_hbm.at[srcK_v.at[g + 3]], rows_b, sem_gb)

    plsc.subcore_barrier()
    pltpu.sync_copy(acc_sh.at[pl.ds(sid * CROWS_PER_TILE, CROWS_PER_TILE)],
                    out_hbm.at[cid, pl.ds(sid * CROWS_PER_TILE,
                                          CROWS_PER_TILE)])


@functools.partial(
    pl.kernel,
    out_type=jax.ShapeDtypeStruct((BATCH, D_FEAT), jnp.float32),
    mesh=_mesh,
    scratch_types=[
        pltpu.VMEM((BATCH // NW,), jnp.int32),
        pltpu.VMEM((BATCH // NW, D_FEAT), jnp.float32),
        pltpu.SemaphoreType.DMA,
    ],
)
def _sc_gather_rows(table_hbm, idx_hbm, out_hbm, idx_v, rows_v, sem):
    b_per_w = BATCH // NW
    base = (lax.axis_index("s") * NC + lax.axis_index("c")) * b_per_w
    pltpu.sync_copy(idx_hbm.at[pl.ds(base, b_per_w)], idx_v)
    pltpu.async_copy(table_hbm.at[idx_v], rows_v, sem).wait()
    pltpu.sync_copy(rows_v, out_hbm.at[pl.ds(base, b_per_w)])


def _tc_scale_body(x_ref, deg_ref, y_ref):
    d = deg_ref[0, :N_NODES, :] + deg_ref[1, :N_NODES, :]
    dis = lax.rsqrt(jnp.maximum(d, 1.0))
    y_ref[...] = x_ref[...] * dis


def _tc_mid_body(p_ref, deg_ref, w_ref, z_ref):
    s = p_ref[0, :N_NODES, :] + p_ref[1, :N_NODES, :]
    d = deg_ref[0, :N_NODES, :] + deg_ref[1, :N_NODES, :]
    dis = lax.rsqrt(jnp.maximum(d, 1.0))
    h = lax.dot_general(s * dis, w_ref[...], (((1,), (0,)), ((), ())),
                        precision=lax.Precision.HIGHEST)
    z_ref[...] = jnp.maximum(h, 0.0) * dis


def _tc_out_body(c_ref, degb_ref, w_ref, o_ref):
    s = c_ref[0, :BATCH, :] + c_ref[1, :BATCH, :]
    d = degb_ref[0] + degb_ref[1]
    dis = lax.rsqrt(jnp.maximum(d, 1.0))
    o_ref[...] = lax.dot_general(s * dis, w_ref[...], (((1,), (0,)), ((), ())),
                                 precision=lax.Precision.HIGHEST)


def kernel(x, W1, W2, edge_index, idx):
    src = edge_index[0]
    dst = edge_index[1]
    pad = E_PAD - N_EDGES
    src_p = jnp.concatenate([src, jnp.zeros((pad,), jnp.int32)])
    dst_p = jnp.concatenate([dst, jnp.full((pad,), N_NODES, jnp.int32)])
    src2d = src_p.reshape(E_PAD // CH, CH)
    dst2d = dst_p.reshape(E_PAD // CH, CH)

    zeros_flat = jnp.zeros((N_PAD,), jnp.float32)
    zeros_blk = jnp.zeros((CH, D_FEAT), jnp.float32)
    sentinv = jnp.full((N_PAD,), BATCH, jnp.int32)
    sentk = jnp.concatenate([jnp.zeros((KCAP_CHUNKS, CH), jnp.int32),
                             jnp.full((KCAP_CHUNKS, CH), BATCH, jnp.int32)])
    W2p = jnp.concatenate(
        [W2, jnp.zeros((D_FEAT, D_FEAT - D_OUT), jnp.float32)], axis=1)

    deg, degB = _sc_degree(dst_p, idx, zeros_flat)
    deg_col = deg.reshape(NC, N_PAD, 1)

    srcK, posK, cnts, posB = _sc_filter(dst_p, src_p, idx, sentinv, sentk)
    posB = posB.reshape(BATCH)

    y = pl.pallas_call(
        _tc_scale_body,
        out_shape=jax.ShapeDtypeStruct((N_NODES, D_FEAT), jnp.float32),
    )(x, deg_col)

    p = _sc_propagate(y, src2d, dst2d, zeros_blk)

    z = pl.pallas_call(
        _tc_mid_body,
        out_shape=jax.ShapeDtypeStruct((N_NODES, D_FEAT), jnp.float32),
    )(p, deg_col, W1)

    c = _sc_prop_compact(z, srcK, posK, cnts, zeros_blk)

    out_c = pl.pallas_call(
        _tc_out_body,
        out_shape=jax.ShapeDtypeStruct((BATCH, D_FEAT), jnp.float32),
    )(c, degB.reshape(NC, BATCH, 1), W2p)

    return _sc_gather_rows(out_c, posB)[:, :D_OUT]


# speedup vs baseline: 2.1650x; 1.2127x over previous
import dataclasses
import functools

import jax
import jax.numpy as jnp
from jax import lax
from jax.experimental import pallas as pl
from jax.experimental.pallas import tpu as pltpu
from jax.experimental.pallas import tpu_sc as plsc

N_NODES = 10000
N_EDGES = 320000
D_FEAT = 128
D_OUT = 64
BATCH = 1024

NC = 2
NS = 16
NW = NC * NS
CH = 128
N_PAD = 10240
ROWS_PER_TILE = N_PAD // NS
E_PAD = 327680
CHUNKS_PER_TILE = E_PAD // (NW * CH)
HROWS_PER_TILE = E_PAD // (NW * 16)

_mesh = plsc.VectorSubcoreMesh(core_axis_name="c", subcore_axis_name="s")


def _sc_params():
    cp = pltpu.CompilerParams()
    if "needs_layout_passes" in pltpu.CompilerParams.__dataclass_fields__:
        cp = dataclasses.replace(cp, needs_layout_passes=False)
    return cp


@functools.partial(
    pl.kernel,
    out_type=(
        jax.ShapeDtypeStruct((NC, N_PAD), jnp.float32),
        jax.ShapeDtypeStruct((NC * BATCH,), jnp.float32),
    ),
    mesh=_mesh,
    compiler_params=_sc_params(),
    scratch_types=[
        pltpu.VMEM((E_PAD // NW,), jnp.int32),
        pltpu.VMEM((N_PAD,), jnp.float32),
        pltpu.VMEM((NS, ROWS_PER_TILE), jnp.float32),
        pltpu.VMEM((ROWS_PER_TILE,), jnp.float32),
        pltpu.VMEM((BATCH // NS,), jnp.int32),
        pltpu.VMEM((BATCH // NS,), jnp.float32),
        pltpu.VMEM_SHARED((NS, N_PAD), jnp.float32),
    ],
)
def _sc_degree(dst_hbm, idx_hbm, zeros_hbm, out_hbm, degb_hbm, dst_v, hist_v,
               colsum_v, deg_v, idxb_v, degb_v, hist_sh):
    cid = lax.axis_index("c")
    sid = lax.axis_index("s")
    w2 = cid * NS + sid
    b_per_t = BATCH // NS
    ET = E_PAD // NW
    pltpu.sync_copy(zeros_hbm, hist_v)
    pltpu.sync_copy(dst_hbm.at[pl.ds(w2 * ET, ET)], dst_v)
    pltpu.sync_copy(idx_hbm.at[pl.ds(sid * b_per_t, b_per_t)], idxb_v)

    @pl.loop(0, ET // 16)
    def _(j):
        d16 = dst_v[pl.ds(j * 16, 16)]
        cnt, last = plsc.scan_count(d16)
        plsc.addupdate_scatter(hist_v, [d16], cnt.astype(jnp.float32),
                               mask=last)

    pltpu.sync_copy(hist_v, hist_sh.at[sid])
    plsc.subcore_barrier()
    pltpu.sync_copy(hist_sh.at[:, pl.ds(sid * ROWS_PER_TILE, ROWS_PER_TILE)],
                    colsum_v)

    @pl.loop(0, ROWS_PER_TILE // 16)
    def _(cc):
        acc = colsum_v[0, pl.ds(cc * 16, 16)]
        for t in range(1, NS):
            acc = acc + colsum_v[t, pl.ds(cc * 16, 16)]
        deg_v[pl.ds(cc * 16, 16)] = acc

    pltpu.sync_copy(deg_v,
                    out_hbm.at[cid, pl.ds(sid * ROWS_PER_TILE, ROWS_PER_TILE)])

    plsc.subcore_barrier()
    pltpu.sync_copy(deg_v, hist_sh.at[0, pl.ds(sid * ROWS_PER_TILE,
                                               ROWS_PER_TILE)])
    plsc.subcore_barrier()
    pltpu.sync_copy(hist_sh.at[0], hist_v)

    @pl.loop(0, b_per_t // 16)
    def _(r2):
        iv = idxb_v[pl.ds(r2 * 16, 16)]
        degb_v[pl.ds(r2 * 16, 16)] = plsc.load_gather(hist_v, [iv])

    pltpu.sync_copy(degb_v,
                    degb_hbm.at[pl.ds(cid * BATCH + sid * b_per_t, b_per_t)])


PH_CHUNKS = CHUNKS_PER_TILE // 2


@functools.partial(
    pl.kernel,
    out_type=jax.ShapeDtypeStruct((NC, N_PAD, D_FEAT), jnp.float32),
    mesh=_mesh,
    scratch_types=[
        pltpu.VMEM((PH_CHUNKS, CH), jnp.int32),
        pltpu.VMEM((PH_CHUNKS, CH), jnp.int32),
        pltpu.VMEM((2 * CH, D_FEAT), jnp.float32),
        pltpu.VMEM_SHARED((N_PAD, D_FEAT), jnp.float32),
        pltpu.SemaphoreType.DMA,
        pltpu.SemaphoreType.DMA,
        pltpu.SemaphoreType.DMA,
        pltpu.SemaphoreType.DMA,
        pltpu.SemaphoreType.DMA,
        pltpu.SemaphoreType.DMA,
    ],
)
def _sc_propagate(h_hbm, src_hbm, dst_hbm, zeros_hbm, out_hbm, src_v, dst_v,
                  rows_v, acc_sh, sem_g0, sem_g1, sem_g2, sem_g3, sem_sa,
                  sem_sb):
    cid = lax.axis_index("c")
    sid = lax.axis_index("s")
    wid = sid * NC + cid
    H = CH // 2
    gsem = (sem_g0, sem_g1, sem_g2, sem_g3)
    pltpu.sync_copy(zeros_hbm, rows_v.at[pl.ds(0, CH)])

    @pl.loop(0, ROWS_PER_TILE // CH)
    def _(k):
        pltpu.sync_copy(rows_v.at[pl.ds(0, CH)],
                        acc_sh.at[pl.ds(sid * ROWS_PER_TILE + k * CH, CH)])

    plsc.subcore_barrier()

    def _gather(g, slot):
        for hh in range(2):
            pltpu.async_copy(
                h_hbm.at[src_v.at[g, pl.ds(hh * H, H)]],
                rows_v.at[pl.ds((2 * slot + hh) * H, H)],
                gsem[2 * slot + hh])

    def _gwait(g, slot):
        for hh in range(2):
            pltpu.make_async_copy(
                h_hbm.at[src_v.at[g, pl.ds(hh * H, H)]],
                rows_v.at[pl.ds((2 * slot + hh) * H, H)],
                gsem[2 * slot + hh]).wait()

    for ph in range(2):
        base = wid * CHUNKS_PER_TILE + ph * PH_CHUNKS
        pltpu.sync_copy(src_hbm.at[pl.ds(base, PH_CHUNKS)], src_v)
        pltpu.sync_copy(dst_hbm.at[pl.ds(base, PH_CHUNKS)], dst_v)
        _gather(0, 0)
        _gather(1, 1)

        @pl.loop(0, PH_CHUNKS, step=2)
        def _(g):
            _gwait(g, 0)
            sca = pltpu.async_copy(rows_v.at[pl.ds(0, CH)],
                                   acc_sh.at[dst_v.at[g]], sem_sa, add=True)
            _gwait(g + 1, 1)
            scb = pltpu.async_copy(rows_v.at[pl.ds(CH, CH)],
                                   acc_sh.at[dst_v.at[g + 1]], sem_sb,
                                   add=True)
            sca.wait()

            @pl.when(g + 2 < PH_CHUNKS)
            def _():
                _gather(g + 2, 0)

            scb.wait()

            @pl.when(g + 3 < PH_CHUNKS)
            def _():
                _gather(g + 3, 1)

    plsc.subcore_barrier()
    pltpu.sync_copy(acc_sh.at[pl.ds(sid * ROWS_PER_TILE, ROWS_PER_TILE)],
                    out_hbm.at[cid, pl.ds(sid * ROWS_PER_TILE, ROWS_PER_TILE)])


KCAP_CHUNKS = 88
KCAP = KCAP_CHUNKS * CH
C_PAD = 1152
CROWS_PER_TILE = C_PAD // NS


@functools.partial(
    pl.kernel,
    out_type=(
        jax.ShapeDtypeStruct((NW, KCAP_CHUNKS, CH), jnp.int32),
        jax.ShapeDtypeStruct((NW, KCAP_CHUNKS, CH), jnp.int32),
        jax.ShapeDtypeStruct((NW * 16,), jnp.int32),
        jax.ShapeDtypeStruct((NW * 32,), jnp.int32),
    ),
    mesh=_mesh,
    compiler_params=_sc_params(),
    scratch_types=[
        pltpu.VMEM((2 * E_PAD // NW,), jnp.int32),
        pltpu.VMEM((N_PAD,), jnp.int32),
        pltpu.VMEM((2 * KCAP_CHUNKS, CH), jnp.int32),
        pltpu.VMEM((BATCH + 64,), jnp.int32),
    ],
)
def _sc_filter(dst_hbm, src_hbm, idx_hbm, sentinv_hbm, sentk_hbm,
               srcK_hbm, posK_hbm, cnts_hbm, posB_hbm,
               sd_v, inv_v, spk_v, misc_v):
    cid = lax.axis_index("c")
    sid = lax.axis_index("s")
    w2 = cid * NS + sid
    ET = E_PAD // NW
    pltpu.sync_copy(dst_hbm.at[pl.ds(w2 * ET, ET)], sd_v.at[pl.ds(0, ET)])
    pltpu.sync_copy(src_hbm.at[pl.ds(w2 * ET, ET)], sd_v.at[pl.ds(ET, ET)])
    pltpu.sync_copy(idx_hbm, misc_v.at[pl.ds(0, BATCH)])
    pltpu.sync_copy(sentinv_hbm, inv_v)
    pltpu.sync_copy(sentk_hbm, spk_v)

    @pl.loop(0, BATCH // 16)
    def _(r):
        v16 = misc_v[pl.ds(r * 16, 16)]
        _, last = plsc.scan_count(v16)
        b16 = r * 16 + lax.iota(jnp.int32, 16)
        plsc.store_scatter(inv_v, [v16], b16, mask=last)

    def body(j, cnt16):
        d16 = sd_v[pl.ds(j * 16, 16)]
        s16 = sd_v[pl.ds(ET + j * 16, 16)]
        p16 = plsc.load_gather(inv_v, [d16])
        m = p16 < BATCH
        offs = plsc.cumsum(m.astype(jnp.int32))
        tgt = cnt16 + offs - 1
        tr = tgt // CH
        tc = tgt - tr * CH
        plsc.store_scatter(spk_v, [tr, tc], s16, mask=m)
        plsc.store_scatter(spk_v, [tr + KCAP_CHUNKS, tc], p16, mask=m)
        return cnt16 + plsc.all_reduce_population_count(m)

    cnt16 = lax.fori_loop(0, ET // 16, body, jnp.zeros((16,), jnp.int32))

    @pl.loop(0, 2)
    def _(r2):
        iv = misc_v[pl.ds(w2 * 32 + r2 * 16, 16)]
        misc_v[pl.ds(BATCH + r2 * 16, 16)] = plsc.load_gather(inv_v, [iv])

    misc_v[pl.ds(BATCH + 32, 16)] = cnt16
    pltpu.sync_copy(spk_v.at[pl.ds(0, KCAP_CHUNKS)], srcK_hbm.at[w2])
    pltpu.sync_copy(spk_v.at[pl.ds(KCAP_CHUNKS, KCAP_CHUNKS)],
                    posK_hbm.at[w2])
    pltpu.sync_copy(misc_v.at[pl.ds(BATCH + 32, 16)],
                    cnts_hbm.at[pl.ds(w2 * 16, 16)])
    pltpu.sync_copy(misc_v.at[pl.ds(BATCH, 32)],
                    posB_hbm.at[pl.ds(w2 * 32, 32)])


@functools.partial(
    pl.kernel,
    out_type=jax.ShapeDtypeStruct((NC, C_PAD, D_FEAT), jnp.float32),
    mesh=_mesh,
    compiler_params=_sc_params(),
    scratch_types=[
        pltpu.VMEM((KCAP_CHUNKS, CH), jnp.int32),
        pltpu.VMEM((KCAP_CHUNKS, CH), jnp.int32),
        pltpu.VMEM((16,), jnp.int32),
        pltpu.VMEM((CH, D_FEAT), jnp.float32),
        pltpu.VMEM((CH, D_FEAT), jnp.float32),
        pltpu.VMEM_SHARED((C_PAD, D_FEAT), jnp.float32),
        pltpu.SemaphoreType.DMA,
        pltpu.SemaphoreType.DMA,
        pltpu.SemaphoreType.DMA,
        pltpu.SemaphoreType.DMA,
    ],
)
def _sc_prop_compact(h_hbm, srcK_hbm, posK_hbm, cnts_hbm, zeros_hbm, out_hbm,
                     srcK_v, posK_v, cnt_v, rows_a, rows_b, acc_sh, sem_ga,
                     sem_gb, sem_sa, sem_sb):
    cid = lax.axis_index("c")
    sid = lax.axis_index("s")
    w2 = cid * NS + sid
    pltpu.sync_copy(zeros_hbm.at[pl.ds(0, CROWS_PER_TILE)],
                    acc_sh.at[pl.ds(sid * CROWS_PER_TILE, CROWS_PER_TILE)])
    pltpu.sync_copy(srcK_hbm.at[w2], srcK_v)
    pltpu.sync_copy(posK_hbm.at[w2], posK_v)
    pltpu.sync_copy(cnts_hbm.at[pl.ds(w2 * 16, 16)], cnt_v)
    plsc.subcore_barrier()

    cnt = jnp.max(cnt_v[...])

    def active(g):
        return g * CH < cnt

    @pl.when(active(0))
    def _():
        pltpu.async_copy(h_hbm.at[srcK_v.at[0]], rows_a, sem_ga)

    @pl.when(active(1))
    def _():
        pltpu.async_copy(h_hbm.at[srcK_v.at[1]], rows_b, sem_gb)

    @pl.loop(0, KCAP_CHUNKS, step=2)
    def _(g):
        @pl.when(active(g))
        def _():
            pltpu.make_async_copy(h_hbm.at[srcK_v.at[g]], rows_a,
                                  sem_ga).wait()
            sca = pltpu.async_copy(rows_a, acc_sh.at[posK_v.at[g]], sem_sa,
                                   add=True)
            sca.wait()

            @pl.when(active(g + 2))
            def _():
                pltpu.async_copy(h_hbm.at[srcK_v.at[g + 2]], rows_a, sem_ga)

        @pl.when(active(g + 1))
        def _():
            pltpu.make_async_copy(h_hbm.at[srcK_v.at[g + 1]], rows_b,
                                  sem_gb).wait()
            scb = pltpu.async_copy(rows_b, acc_sh.at[posK_v.at[g + 1]], sem_sb,
                                   add=True)
            scb.wait()

            @pl.when(active(g + 3))
            def _():
                pltpu.async_copy(h_hbm.at[srcK_v.at[g + 3]], rows_b, sem_gb)

    plsc.subcore_barrier()
    pltpu.sync_copy(acc_sh.at[pl.ds(sid * CROWS_PER_TILE, CROWS_PER_TILE)],
                    out_hbm.at[cid, pl.ds(sid * CROWS_PER_TILE,
                                          CROWS_PER_TILE)])


@functools.partial(
    pl.kernel,
    out_type=jax.ShapeDtypeStruct((BATCH, D_FEAT), jnp.float32),
    mesh=_mesh,
    scratch_types=[
        pltpu.VMEM((BATCH // NW,), jnp.int32),
        pltpu.VMEM((BATCH // NW, D_FEAT), jnp.float32),
        pltpu.SemaphoreType.DMA,
    ],
)
def _sc_gather_rows(table_hbm, idx_hbm, out_hbm, idx_v, rows_v, sem):
    b_per_w = BATCH // NW
    base = (lax.axis_index("s") * NC + lax.axis_index("c")) * b_per_w
    pltpu.sync_copy(idx_hbm.at[pl.ds(base, b_per_w)], idx_v)
    pltpu.async_copy(table_hbm.at[idx_v], rows_v, sem).wait()
    pltpu.sync_copy(rows_v, out_hbm.at[pl.ds(base, b_per_w)])


def _tc_scale_body(x_ref, deg_ref, y_ref):
    d = deg_ref[0, :N_NODES, :] + deg_ref[1, :N_NODES, :]
    dis = lax.rsqrt(jnp.maximum(d, 1.0))
    y_ref[...] = x_ref[...] * dis


def _tc_mid_body(p_ref, deg_ref, w_ref, z_ref):
    s = p_ref[0, :N_NODES, :] + p_ref[1, :N_NODES, :]
    d = deg_ref[0, :N_NODES, :] + deg_ref[1, :N_NODES, :]
    dis = lax.rsqrt(jnp.maximum(d, 1.0))
    h = lax.dot_general(s * dis, w_ref[...], (((1,), (0,)), ((), ())),
                        precision=lax.Precision.HIGHEST)
    z_ref[...] = jnp.maximum(h, 0.0) * dis


def _tc_out_body(c_ref, degb_ref, w_ref, o_ref):
    s = c_ref[0, :BATCH, :] + c_ref[1, :BATCH, :]
    d = degb_ref[0] + degb_ref[1]
    dis = lax.rsqrt(jnp.maximum(d, 1.0))
    o_ref[...] = lax.dot_general(s * dis, w_ref[...], (((1,), (0,)), ((), ())),
                                 precision=lax.Precision.HIGHEST)


def kernel(x, W1, W2, edge_index, idx):
    src = edge_index[0]
    dst = edge_index[1]
    pad = E_PAD - N_EDGES
    src_p = jnp.concatenate([src, jnp.zeros((pad,), jnp.int32)])
    dst_p = jnp.concatenate([dst, jnp.full((pad,), N_NODES, jnp.int32)])
    src2d = src_p.reshape(E_PAD // CH, CH)
    dst2d = dst_p.reshape(E_PAD // CH, CH)

    zeros_flat = jnp.zeros((N_PAD,), jnp.float32)
    zeros_blk = jnp.zeros((CH, D_FEAT), jnp.float32)
    sentinv = jnp.full((N_PAD,), BATCH, jnp.int32)
    sentk = jnp.concatenate([jnp.zeros((KCAP_CHUNKS, CH), jnp.int32),
                             jnp.full((KCAP_CHUNKS, CH), BATCH, jnp.int32)])
    W2p = jnp.concatenate(
        [W2, jnp.zeros((D_FEAT, D_FEAT - D_OUT), jnp.float32)], axis=1)

    deg, degB = _sc_degree(dst_p, idx, zeros_flat)
    deg_col = deg.reshape(NC, N_PAD, 1)

    srcK, posK, cnts, posB = _sc_filter(dst_p, src_p, idx, sentinv, sentk)
    posB = posB.reshape(BATCH)

    y = pl.pallas_call(
        _tc_scale_body,
        out_shape=jax.ShapeDtypeStruct((N_NODES, D_FEAT), jnp.float32),
    )(x, deg_col)

    y, srcK, posK, cnts, posB = lax.optimization_barrier(
        (y, srcK, posK, cnts, posB))

    p = _sc_propagate(y, src2d, dst2d, zeros_blk)

    z = pl.pallas_call(
        _tc_mid_body,
        out_shape=jax.ShapeDtypeStruct((N_NODES, D_FEAT), jnp.float32),
    )(p, deg_col, W1)

    c = _sc_prop_compact(z, srcK, posK, cnts, zeros_blk)

    out_c = pl.pallas_call(
        _tc_out_body,
        out_shape=jax.ShapeDtypeStruct((BATCH, D_FEAT), jnp.float32),
    )(c, degB.reshape(NC, BATCH, 1), W2p)

    return _sc_gather_rows(out_c, posB)[:, :D_OUT]
